# baseline (speedup 1.0000x reference)
import math

import jax
import jax.numpy as jnp
from jax import lax
from jax.experimental import pallas as pl
from jax.experimental.pallas import tpu as pltpu
from jax.experimental.pallas import tpu_sc as plsc

D_MODEL = 1024
SCALE = math.sqrt(D_MODEL)

_INFO = plsc.get_sparse_core_info()
_NC = _INFO.num_cores
_NS = _INFO.num_subcores
_NW = _NC * _NS
_L = _INFO.num_lanes

_B = 16384
_PER_W = _B // _NW
_C = 32
_NCHUNK = _PER_W // _C
_NBUF = 2


def _emb_body(x_hbm, table_hbm, out_hbm, idx_v, buf, gsem0, gsem1,
              wsem0, wsem1):
    gsem = (gsem0, gsem1)
    wsem = (wsem0, wsem1)
    wid = lax.axis_index("s") * _NC + lax.axis_index("c")
    base = wid * _PER_W

    pltpu.sync_copy(x_hbm.at[pl.ds(base, _PER_W)], idx_v)

    for b in range(_NBUF):
        pltpu.async_copy(
            table_hbm.at[idx_v.at[pl.ds(b * _C, _C)]], buf.at[b], gsem[b])

    @pl.loop(0, _NCHUNK, step=_NBUF)
    def _outer(g0):
        for b in range(_NBUF):
            g = g0 + b
            pltpu.make_async_copy(
                table_hbm.at[pl.ds(0, _C)], buf.at[b], gsem[b]).wait()

            @plsc.parallel_loop(0, _C, unroll=2)
            def _row(r):
                for j in range(D_MODEL // _L):
                    sl = pl.ds(j * _L, _L)
                    buf[b, r, sl] = buf[b, r, sl] * SCALE

            pltpu.async_copy(
                buf.at[b], out_hbm.at[pl.ds(base + g * _C, _C)], wsem[b])

            @pl.when(g + _NBUF < _NCHUNK)
            def _():
                pltpu.make_async_copy(
                    buf.at[b], out_hbm.at[pl.ds(0, _C)], wsem[b]).wait()
                pltpu.async_copy(
                    table_hbm.at[idx_v.at[pl.ds((g + _NBUF) * _C, _C)]],
                    buf.at[b], gsem[b])

    for b in range(_NBUF):
        pltpu.make_async_copy(
            buf.at[b], out_hbm.at[pl.ds(0, _C)], wsem[b]).wait()


_emb = pl.kernel(
    _emb_body,
    out_type=jax.ShapeDtypeStruct((_B, D_MODEL), jnp.float32),
    mesh=plsc.VectorSubcoreMesh(core_axis_name="c", subcore_axis_name="s"),
    scratch_types=[
        pltpu.VMEM((_PER_W,), jnp.int32),
        pltpu.VMEM((_NBUF, _C, D_MODEL), jnp.float32),
        pltpu.SemaphoreType.DMA,
        pltpu.SemaphoreType.DMA,
        pltpu.SemaphoreType.DMA,
        pltpu.SemaphoreType.DMA,
    ],
)


@jax.jit
def kernel(x, table):
    xi = x.reshape(-1).astype(jnp.int32)
    out = _emb(xi, table)
    return out.reshape(x.shape + (D_MODEL,))


# speedup vs baseline: 1.0907x; 1.0907x over previous
import math

import jax
import jax.numpy as jnp
from jax import lax
from jax.experimental import pallas as pl
from jax.experimental.pallas import tpu as pltpu
from jax.experimental.pallas import tpu_sc as plsc

D_MODEL = 1024
SCALE = math.sqrt(D_MODEL)

_INFO = plsc.get_sparse_core_info()
_NC = _INFO.num_cores
_NS = _INFO.num_subcores
_NW = _NC * _NS
_L = _INFO.num_lanes

_B = 16384
_PER_W = _B // _NW
_C = 32
_NCHUNK = _PER_W // _C
_NBUF = 3


def _emb_body(x_hbm, table_hbm, out_hbm, idx_v, buf, gsem0, gsem1, gsem2,
              wsem0, wsem1, wsem2):
    gsem = (gsem0, gsem1, gsem2)
    wsem = (wsem0, wsem1, wsem2)
    wid = lax.axis_index("s") * _NC + lax.axis_index("c")
    base = wid * _PER_W

    pltpu.sync_copy(x_hbm.at[pl.ds(base, _PER_W)], idx_v)

    for b in range(_NBUF):
        pltpu.async_copy(
            table_hbm.at[idx_v.at[pl.ds(b * _C, _C)]], buf.at[b], gsem[b])

    def _step(h, b):
        pltpu.make_async_copy(
            table_hbm.at[pl.ds(0, _C)], buf.at[b], gsem[b]).wait()

        @plsc.parallel_loop(0, _C)
        def _row(r):
            for j in range(D_MODEL // _L):
                sl = pl.ds(j * _L, _L)
                buf[b, r, sl] = buf[b, r, sl] * SCALE

        pltpu.async_copy(
            buf.at[b], out_hbm.at[pl.ds(base + h * _C, _C)], wsem[b])

    def _refill(h, bp):
        pltpu.make_async_copy(
            buf.at[bp], out_hbm.at[pl.ds(0, _C)], wsem[bp]).wait()
        pltpu.async_copy(
            table_hbm.at[idx_v.at[pl.ds((h + 2) * _C, _C)]],
            buf.at[bp], gsem[bp])

    @pl.loop(0, _NCHUNK - 1, step=_NBUF)
    def _outer(g0):
        for db in range(_NBUF):
            h = g0 + db
            b = db
            _step(h, b)

            @pl.when((h >= 1) & (h + 2 < _NCHUNK))
            def _():
                _refill(h, (db + 2) % _NBUF)

    _step(_NCHUNK - 1, (_NCHUNK - 1) % _NBUF)

    for b in range(_NBUF):
        pltpu.make_async_copy(
            buf.at[b], out_hbm.at[pl.ds(0, _C)], wsem[b]).wait()


_emb = pl.kernel(
    _emb_body,
    out_type=jax.ShapeDtypeStruct((_B, D_MODEL), jnp.float32),
    mesh=plsc.VectorSubcoreMesh(core_axis_name="c", subcore_axis_name="s"),
    scratch_types=[
        pltpu.VMEM((_PER_W,), jnp.int32),
        pltpu.VMEM((_NBUF, _C, D_MODEL), jnp.float32),
        pltpu.SemaphoreType.DMA,
        pltpu.SemaphoreType.DMA,
        pltpu.SemaphoreType.DMA,
        pltpu.SemaphoreType.DMA,
        pltpu.SemaphoreType.DMA,
        pltpu.SemaphoreType.DMA,
    ],
)


@jax.jit
def kernel(x, table):
    xi = x.reshape(-1).astype(jnp.int32)
    out = _emb(xi, table)
    return out.reshape(x.shape + (D_MODEL,))


# speedup vs baseline: 1.1103x; 1.0179x over previous
import math

import jax
import jax.numpy as jnp
from jax import lax
from jax.experimental import pallas as pl
from jax.experimental.pallas import tpu as pltpu
from jax.experimental.pallas import tpu_sc as plsc

D_MODEL = 1024
SCALE = math.sqrt(D_MODEL)

_INFO = plsc.get_sparse_core_info()
_NC = _INFO.num_cores
_NS = _INFO.num_subcores
_NW = _NC * _NS
_L = _INFO.num_lanes

_B = 16384
_PER_W = _B // _NW
_C = 32
_NCHUNK = _PER_W // _C
_NBUF = 3


def _emb_body(x_hbm, table_hbm, out_hbm, idx_v, buf, gsem0, gsem1, gsem2,
              wsem0, wsem1, wsem2):
    gsem = (gsem0, gsem1, gsem2)
    wsem = (wsem0, wsem1, wsem2)
    wid = lax.axis_index("s") * _NC + lax.axis_index("c")
    base = wid * _PER_W

    pltpu.sync_copy(x_hbm.at[pl.ds(base, _C)], idx_v.at[pl.ds(0, _C)])
    pltpu.async_copy(
        table_hbm.at[idx_v.at[pl.ds(0, _C)]], buf.at[0], gsem[0])
    pltpu.sync_copy(x_hbm.at[pl.ds(base + _C, _PER_W - _C)],
                    idx_v.at[pl.ds(_C, _PER_W - _C)])
    for b in range(1, _NBUF):
        pltpu.async_copy(
            table_hbm.at[idx_v.at[pl.ds(b * _C, _C)]], buf.at[b], gsem[b])

    def _step(h, b):
        pltpu.make_async_copy(
            table_hbm.at[pl.ds(0, _C)], buf.at[b], gsem[b]).wait()

        @plsc.parallel_loop(0, _C)
        def _row(r):
            for j in range(D_MODEL // _L):
                sl = pl.ds(j * _L, _L)
                buf[b, r, sl] = buf[b, r, sl] * SCALE

        pltpu.async_copy(
            buf.at[b], out_hbm.at[pl.ds(base + h * _C, _C)], wsem[b])

    def _refill(h, bp):
        pltpu.make_async_copy(
            buf.at[bp], out_hbm.at[pl.ds(0, _C)], wsem[bp]).wait()
        pltpu.async_copy(
            table_hbm.at[idx_v.at[pl.ds((h + 2) * _C, _C)]],
            buf.at[bp], gsem[bp])

    @pl.loop(0, _NCHUNK - 1, step=_NBUF)
    def _outer(g0):
        for db in range(_NBUF):
            h = g0 + db
            b = db
            _step(h, b)

            @pl.when((h >= 1) & (h + 2 < _NCHUNK))
            def _():
                _refill(h, (db + 2) % _NBUF)

    _step(_NCHUNK - 1, (_NCHUNK - 1) % _NBUF)

    for b in range(_NBUF):
        pltpu.make_async_copy(
            buf.at[b], out_hbm.at[pl.ds(0, _C)], wsem[b]).wait()


_emb = pl.kernel(
    _emb_body,
    out_type=jax.ShapeDtypeStruct((_B, D_MODEL), jnp.float32),
    mesh=plsc.VectorSubcoreMesh(core_axis_name="c", subcore_axis_name="s"),
    scratch_types=[
        pltpu.VMEM((_PER_W,), jnp.int32),
        pltpu.VMEM((_NBUF, _C, D_MODEL), jnp.float32),
        pltpu.SemaphoreType.DMA,
        pltpu.SemaphoreType.DMA,
        pltpu.SemaphoreType.DMA,
        pltpu.SemaphoreType.DMA,
        pltpu.SemaphoreType.DMA,
        pltpu.SemaphoreType.DMA,
    ],
)


@jax.jit
def kernel(x, table):
    xi = x.reshape(-1).astype(jnp.int32)
    out = _emb(xi, table)
    return out.reshape(x.shape + (D_MODEL,))
